# baseline (speedup 1.0000x reference)
import functools

import jax
import jax.numpy as jnp
from jax import lax
from jax.experimental import pallas as pl
from jax.experimental.pallas import tpu as pltpu
from jax.experimental.pallas import tpu_sc as plsc

DIM = 64
BATCH = 16384
HIST = 50

B = BATCH * HIST
NW = 32
CHUNK = 128
N_CHUNKS = B // CHUNK
C_PER_W = N_CHUNKS // NW
BT = BATCH // CHUNK


def _make_gather():
    mesh = plsc.VectorSubcoreMesh(core_axis_name="c", subcore_axis_name="s")

    @functools.partial(
        pl.kernel,
        mesh=mesh,
        out_type=jax.ShapeDtypeStruct((HIST, DIM // 8, BT, 1024),
                                      jnp.float32),
        scratch_types=[
            pltpu.VMEM((C_PER_W, CHUNK), jnp.int32),
            [pltpu.VMEM((CHUNK, DIM), jnp.float32) for _ in range(2)],
            [pltpu.VMEM((8 * 1024,), jnp.float32) for _ in range(2)],
            [pltpu.SemaphoreType.DMA for _ in range(2)],
            [pltpu.SemaphoreType.DMA for _ in range(2)],
        ],
        compiler_params=pltpu.CompilerParams(use_tc_tiling_on_sc=False,
                                             needs_layout_passes=False),
    )
    def gather_kernel(table_hbm, idx_hbm, out_hbm, idx_v, rows, tbuf,
                      gsem, wsem):
        wid = lax.axis_index("s") * 2 + lax.axis_index("c")
        chunk0 = wid * C_PER_W
        pltpu.sync_copy(idx_hbm.at[pl.ds(chunk0, C_PER_W)], idx_v)

        lane = lax.broadcasted_iota(jnp.int32, (16,), 0)
        dbases = [(lane + 16 * k) * CHUNK for k in range(DIM // 16)]

        def fire(c, p):
            pltpu.async_copy(table_hbm.at[idx_v.at[c]], rows[p], gsem[p])

        def drain_gather(c, p):
            pltpu.make_async_copy(table_hbm.at[idx_v.at[c]], rows[p],
                                  gsem[p]).wait()

        def out_tiles(cg):
            h = cg // BT
            tc = cg % BT
            return [out_hbm.at[h, tr, tc] for tr in range(8)]

        fire(0, 0)

        def pair_body(c2, _):
            for p in range(2):
                c = 2 * c2 + p
                drain_gather(c, p)

                @pl.when(c + 1 < C_PER_W)
                def _():
                    fire(c + 1, 1 - p)

                @pl.when(c2 >= 1)
                def _():
                    for tr in range(8):
                        pltpu.make_async_copy(
                            tbuf[p].at[pl.ds(tr * 1024, 1024)],
                            out_tiles(chunk0 + c)[tr], wsem[p]).wait()

                def bloop(g, _):
                    b0 = g * 16
                    dbg = [dbases[k] + b0 for k in range(DIM // 16)]
                    for bb in range(16):
                        for k in range(DIM // 16):
                            v = rows[p][b0 + bb, pl.ds(16 * k, 16)]
                            plsc.store_scatter(tbuf[p], [dbg[k] + bb], v)
                    return 0

                lax.fori_loop(0, CHUNK // 16, bloop, 0)

                tiles = out_tiles(chunk0 + c)
                for tr in range(8):
                    pltpu.async_copy(tbuf[p].at[pl.ds(tr * 1024, 1024)],
                                     tiles[tr], wsem[p])
            return 0

        lax.fori_loop(0, C_PER_W // 2, pair_body, 0)

        for p in range(2):
            c = C_PER_W - 2 + p
            for tr in range(8):
                pltpu.make_async_copy(
                    tbuf[p].at[pl.ds(tr * 1024, 1024)],
                    out_tiles(chunk0 + c)[tr], wsem[p]).wait()

    return gather_kernel


_gather = _make_gather()


def kernel(data, ivectors, ovectors):
    idx = data.T.reshape(N_CHUNKS, CHUNK).astype(jnp.int32)
    o5 = _gather(ivectors, idx)
    out = (o5.reshape(HIST, 8, BT, 8, CHUNK)
           .transpose(2, 4, 0, 1, 3)
           .reshape(BATCH * HIST // HIST, HIST, DIM))
    return out


# speedup vs baseline: 1.4712x; 1.4712x over previous
import functools

import jax
import jax.numpy as jnp
from jax import lax
from jax.experimental import pallas as pl
from jax.experimental.pallas import tpu as pltpu
from jax.experimental.pallas import tpu_sc as plsc

DIM = 64
BATCH = 16384
HIST = 50

B = BATCH * HIST
NW = 32
CHUNK = 128
N_CHUNKS = B // CHUNK
C_PER_W = N_CHUNKS // NW
BT = BATCH // CHUNK


def _make_gather():
    mesh = plsc.VectorSubcoreMesh(core_axis_name="c", subcore_axis_name="s")

    @functools.partial(
        pl.kernel,
        mesh=mesh,
        out_type=jax.ShapeDtypeStruct((HIST, DIM // 8, BT, 1024),
                                      jnp.float32),
        scratch_types=[
            pltpu.VMEM((C_PER_W, CHUNK), jnp.int32),
            [pltpu.VMEM((CHUNK, DIM), jnp.float32) for _ in range(2)],
            [pltpu.VMEM((8 * 1024,), jnp.float32) for _ in range(2)],
            [pltpu.SemaphoreType.DMA for _ in range(2)],
            [pltpu.SemaphoreType.DMA for _ in range(2)],
        ],
        compiler_params=pltpu.CompilerParams(use_tc_tiling_on_sc=False,
                                             needs_layout_passes=False),
    )
    def gather_kernel(table_hbm, idx_hbm, out_hbm, idx_v, rows, tbuf,
                      gsem, wsem):
        wid = lax.axis_index("s") * 2 + lax.axis_index("c")
        chunk0 = wid * C_PER_W
        pltpu.sync_copy(idx_hbm.at[pl.ds(chunk0, C_PER_W)], idx_v)

        lane = lax.broadcasted_iota(jnp.int32, (16,), 0)
        jmods = [(lane + r) % 16 for r in range(16)]
        jm7l = [jmods[r] * CHUNK + lane for r in range(16)]

        def fire(c, p):
            pltpu.async_copy(table_hbm.at[idx_v.at[c]], rows[p], gsem[p])

        def drain_gather(c, p):
            pltpu.make_async_copy(table_hbm.at[idx_v.at[c]], rows[p],
                                  gsem[p]).wait()

        def out_tiles(cg):
            h = cg // BT
            tc = cg % BT
            return [out_hbm.at[h, tr, tc] for tr in range(8)]

        fire(0, 0)

        def pair_body(c2, _):
            for p in range(2):
                c = 2 * c2 + p
                drain_gather(c, p)

                @pl.when(c + 1 < C_PER_W)
                def _():
                    fire(c + 1, 1 - p)

                @pl.when(c2 >= 1)
                def _():
                    for tr in range(8):
                        pltpu.make_async_copy(
                            tbuf[p].at[pl.ds(tr * 1024, 1024)],
                            out_tiles(chunk0 + c)[tr], wsem[p]).wait()

                def bloop(g, _):
                    b0 = g * 16
                    bvec = b0 + lane
                    for d0 in range(0, DIM, 16):
                        for r in range(16):
                            v = plsc.load_gather(
                                rows[p], [bvec, jmods[r] + d0])
                            plsc.store_scatter(
                                tbuf[p], [jm7l[r] + (d0 * CHUNK + b0)], v)
                    return 0

                lax.fori_loop(0, CHUNK // 16, bloop, 0)

                tiles = out_tiles(chunk0 + c)
                for tr in range(8):
                    pltpu.async_copy(tbuf[p].at[pl.ds(tr * 1024, 1024)],
                                     tiles[tr], wsem[p])
            return 0

        lax.fori_loop(0, C_PER_W // 2, pair_body, 0)

        for p in range(2):
            c = C_PER_W - 2 + p
            for tr in range(8):
                pltpu.make_async_copy(
                    tbuf[p].at[pl.ds(tr * 1024, 1024)],
                    out_tiles(chunk0 + c)[tr], wsem[p]).wait()

    return gather_kernel


_gather = _make_gather()


def kernel(data, ivectors, ovectors):
    idx = data.T.reshape(N_CHUNKS, CHUNK).astype(jnp.int32)
    o5 = _gather(ivectors, idx)
    out = (o5.reshape(HIST, 8, BT, 8, CHUNK)
           .transpose(2, 4, 0, 1, 3)
           .reshape(BATCH * HIST // HIST, HIST, DIM))
    return out
